# baseline (speedup 1.0000x reference)
import functools

import jax
import jax.numpy as jnp
from jax import lax
from jax.experimental import pallas as pl
from jax.experimental.pallas import tpu as pltpu
from jax.experimental.pallas import tpu_sc as plsc

NC = 2
NS = 16
NW = NC * NS
LANES = 16
RING = 4


@functools.lru_cache(maxsize=None)
def _build(B, S, D):
    s_per_w = S // NW
    T = 4
    if s_per_w % (T * RING):
        raise NotImplementedError(s_per_w)
    NT = s_per_w // T
    NVD = D // LANES

    mesh = plsc.VectorSubcoreMesh(core_axis_name="c", subcore_axis_name="s")

    @functools.partial(
        pl.kernel,
        mesh=mesh,
        out_type=jax.ShapeDtypeStruct((B, S, D), jnp.float32),
        scratch_types=[
            pltpu.VMEM((RING, B, T, D), jnp.float32),
            pltpu.VMEM((RING, T, D), jnp.float32),
            pltpu.SemaphoreType.DMA((RING,)),
            pltpu.SemaphoreType.DMA((RING,)),
            pltpu.SemaphoreType.DMA((RING,)),
        ],
    )
    def k(x_hbm, emb_hbm, out_hbm, xbuf, ebuf, sem_x, sem_e, sem_o):
        wid = lax.axis_index("s") * NC + lax.axis_index("c")
        s0 = wid * s_per_w

        def start_x(g, sl):
            for b in range(B):
                pltpu.async_copy(
                    x_hbm.at[b, pl.ds(s0 + g * T, T)], xbuf.at[sl, b],
                    sem_x.at[sl])

        def start_e(g, sl):
            return pltpu.async_copy(
                emb_hbm.at[pl.ds(s0 + g * T, T)], ebuf.at[sl], sem_e.at[sl])

        def start_o(g, sl):
            return pltpu.async_copy(
                xbuf.at[sl], out_hbm.at[:, pl.ds(s0 + g * T, T)], sem_o.at[sl])

        def wait_o(sl):
            pltpu.make_async_copy(
                xbuf.at[sl], out_hbm.at[:, pl.ds(s0, T)], sem_o.at[sl]).wait()

        def wait_in(sl):
            for b in range(B):
                pltpu.make_async_copy(
                    x_hbm.at[b, pl.ds(s0, T)], xbuf.at[sl, b],
                    sem_x.at[sl]).wait()
            pltpu.make_async_copy(
                emb_hbm.at[pl.ds(s0, T)], ebuf.at[sl], sem_e.at[sl]).wait()

        def adds(sl):
            @pl.loop(0, T)
            def _row(r, sl=sl):
                @pl.loop(0, NVD, unroll=8)
                def _add(i):
                    o = pl.ds(i * LANES, LANES)
                    v = ebuf[sl, r, o]
                    for b in range(B):
                        plsc.addupdate(xbuf.at[sl, b, r, o], v)

        for g in range(2):
            start_x(g, g)
            start_e(g, g)

        @pl.loop(0, NT, step=RING)
        def _iter(g0):
            for j in range(RING):
                g = g0 + j
                sl = j
                sl2 = (j + 2) % RING
                with jax.named_scope("ring_wait"):
                    @pl.when(g >= 2)
                    def _():
                        wait_o(sl2)
                    @pl.when(g + 2 < NT)
                    def _():
                        start_x(g + 2, sl2)
                        start_e(g + 2, sl2)
                    wait_in(sl)
                with jax.named_scope("adds"):
                    adds(sl)
                start_o(g, sl)

        for g in range(NT - 2, NT):
            wait_o(g % RING)

    return k


def kernel(x, emb_table):
    B, S, D = x.shape
    return _build(B, S, D)(x, emb_table)


# speedup vs baseline: 1.0072x; 1.0072x over previous
import functools

import jax
import jax.numpy as jnp
from jax import lax
from jax.experimental import pallas as pl
from jax.experimental.pallas import tpu as pltpu
from jax.experimental.pallas import tpu_sc as plsc

NC = 2
NS = 16
NW = NC * NS
LANES = 16
RING = 4


@functools.lru_cache(maxsize=None)
def _build(B, S, D):
    s_per_w = S // NW
    T = 4
    if s_per_w % (T * RING):
        raise NotImplementedError(s_per_w)
    NT = s_per_w // T
    NVD = D // LANES

    mesh = plsc.VectorSubcoreMesh(core_axis_name="c", subcore_axis_name="s")

    @functools.partial(
        pl.kernel,
        mesh=mesh,
        out_type=jax.ShapeDtypeStruct((B, S, D), jnp.float32),
        scratch_types=[
            pltpu.VMEM((RING, B, T, D), jnp.float32),
            pltpu.VMEM((RING, T, D), jnp.float32),
            pltpu.SemaphoreType.DMA((RING,)),
            pltpu.SemaphoreType.DMA((RING,)),
            pltpu.SemaphoreType.DMA((RING,)),
        ],
    )
    def k(x_hbm, emb_hbm, out_hbm, xbuf, ebuf, sem_x, sem_e, sem_o):
        wid = lax.axis_index("s") * NC + lax.axis_index("c")
        s0 = wid * s_per_w

        def start_x(g, sl):
            return pltpu.async_copy(
                x_hbm.at[:, pl.ds(s0 + g * T, T)], xbuf.at[sl], sem_x.at[sl])

        def start_e(g, sl):
            return pltpu.async_copy(
                emb_hbm.at[pl.ds(s0 + g * T, T)], ebuf.at[sl], sem_e.at[sl])

        def start_o(g, sl):
            return pltpu.async_copy(
                xbuf.at[sl], out_hbm.at[:, pl.ds(s0 + g * T, T)], sem_o.at[sl])

        def wait_o(sl):
            pltpu.make_async_copy(
                xbuf.at[sl], out_hbm.at[:, pl.ds(s0, T)], sem_o.at[sl]).wait()

        def wait_in(sl):
            pltpu.make_async_copy(
                x_hbm.at[:, pl.ds(s0, T)], xbuf.at[sl], sem_x.at[sl]).wait()
            pltpu.make_async_copy(
                emb_hbm.at[pl.ds(s0, T)], ebuf.at[sl], sem_e.at[sl]).wait()

        def adds(sl):
            @pl.loop(0, T)
            def _row(r, sl=sl):
                @pl.loop(0, NVD, unroll=8)
                def _add(i):
                    o = pl.ds(i * LANES, LANES)
                    v = ebuf[sl, r, o]
                    for b in range(B):
                        plsc.addupdate(xbuf.at[sl, b, r, o], v)

        for g in range(2):
            start_x(g, g)
            start_e(g, g)

        @pl.loop(0, NT, step=RING)
        def _iter(g0):
            for j in range(RING):
                g = g0 + j
                sl = j
                sl2 = (j + 2) % RING

                @pl.when(g >= 2)
                def _():
                    wait_o(sl2)

                @pl.when(g + 2 < NT)
                def _():
                    start_x(g + 2, sl2)
                    start_e(g + 2, sl2)

                wait_in(sl)
                adds(sl)
                start_o(g, sl)

        for g in range(NT - 2, NT):
            wait_o(g % RING)

    return k


def kernel(x, emb_table):
    B, S, D = x.shape
    return _build(B, S, D)(x, emb_table)
